# baseline (speedup 1.0000x reference)
import functools

import jax
import jax.numpy as jnp
from jax import lax
from jax.experimental import pallas as pl
from jax.experimental.pallas import tpu as pltpu
from jax.experimental.pallas import tpu_sc as plsc

_N = 10000
_C = 256
_E = 160000
_NC = 2
_NS = 16
_K = 128
_CH = 80
_EPAD = _NS * _CH * _K
_NPAD = 10112
_RPT = _NPAD // _NS
_H = _C // 2
_BN = 1000
_PSH = 14

_mesh = plsc.VectorSubcoreMesh(core_axis_name="c", subcore_axis_name="s")


def _fill_f32(buf, nrows, value):
    v = jnp.full((16,), value, jnp.float32)

    def body(i, carry):
        buf[i // 8, pl.ds((i % 8) * 16, 16)] = v
        return carry

    lax.fori_loop(0, nrows * 8, body, 0)


def _zero_own_rows(buf, acc_sh, r0):
    _fill_f32(buf, _K, 0.0)
    for off in range(0, _RPT - 120, _K):
        pltpu.sync_copy(buf, acc_sh.at[pl.ds(r0 + off, _K)])
    pltpu.sync_copy(buf.at[pl.ds(0, 120)],
                    acc_sh.at[pl.ds(r0 + _RPT - 120, 120)])


@functools.partial(
    pl.kernel, mesh=_mesh,
    out_type=jax.ShapeDtypeStruct((_NC, _NPAD, _H), jnp.float32),
    scratch_types=[
        pltpu.VMEM((_CH, _K), jnp.int32),
        pltpu.VMEM((1, _K), jnp.int32),
        pltpu.VMEM((_K, _H), jnp.float32),
        pltpu.VMEM_SHARED((_NPAD, _H), jnp.float32),
    ],
)
def _deg_kernel(pk_hbm, out_hbm, pk_v, di_v, ones_v, acc_sh):
    c = lax.axis_index("c")
    s = lax.axis_index("s")
    pltpu.sync_copy(pk_hbm.at[c, s], pk_v)
    r0 = s * _RPT
    _zero_own_rows(ones_v, acc_sh, r0)
    _fill_f32(ones_v, _K, 1.0)
    plsc.subcore_barrier()

    def body(j, carry):
        for u in range(8):
            sl = pl.ds(u * 16, 16)
            di_v[0, sl] = pk_v[j, sl] & ((1 << _PSH) - 1)
        pltpu.sync_copy(ones_v, acc_sh.at[di_v.at[0]], add=True)
        return carry

    lax.fori_loop(0, _CH, body, 0)
    plsc.subcore_barrier()
    pltpu.sync_copy(acc_sh.at[pl.ds(r0, _RPT)], out_hbm.at[c, pl.ds(r0, _RPT)])


@functools.partial(
    pl.kernel, mesh=_mesh,
    out_type=(
        jax.ShapeDtypeStruct((_NC, _NPAD, _H), jnp.float32),
        jax.ShapeDtypeStruct((_NC, _NPAD, _H), jnp.float32),
    ),
    scratch_types=[
        pltpu.VMEM((_CH, _K), jnp.int32),
        pltpu.VMEM((2, _K), jnp.int32),
        pltpu.VMEM((2, _K), jnp.int32),
        pltpu.VMEM((_K, _H), jnp.float32),
        pltpu.VMEM((_K, _H), jnp.float32),
        pltpu.VMEM_SHARED((_NPAD, _H), jnp.float32),
        pltpu.SemaphoreType.DMA,
        pltpu.SemaphoreType.DMA,
    ],
)
def _agg2_kernel(h0_hbm, pk0_hbm, h1_hbm, pk1_hbm, out0_hbm, out1_hbm,
                 pk_v, si_v, di_v, buf, buf1, acc_sh, sem, sem1):
    c = lax.axis_index("c")
    s = lax.axis_index("s")
    r0 = s * _RPT

    def unpack(j, slot):
        for u in range(8):
            sl = pl.ds(u * 16, 16)
            pk = pk_v[j, sl]
            si_v[slot, sl] = lax.shift_right_logical(pk, _PSH)
            di_v[slot, sl] = pk & ((1 << _PSH) - 1)

    for h_hbm, pk_hbm, out_hbm in ((h0_hbm, pk0_hbm, out0_hbm),
                                   (h1_hbm, pk1_hbm, out1_hbm)):
        hh = h_hbm.at[c]
        pltpu.sync_copy(pk_hbm.at[s], pk_v)
        _zero_own_rows(buf, acc_sh, r0)
        plsc.subcore_barrier()
        unpack(0, 0)
        pltpu.async_copy(hh.at[si_v.at[0]], buf, sem)

        def body(t, carry):
            g = 2 * t
            unpack(g + 1, 1)
            pltpu.async_copy(hh.at[si_v.at[1]], buf1, sem1)
            pltpu.make_async_copy(hh.at[si_v.at[0]], buf, sem).wait()
            pltpu.sync_copy(buf, acc_sh.at[di_v.at[0]], add=True)

            @pl.when(g + 2 < _CH)
            def _():
                unpack(g + 2, 0)
                pltpu.async_copy(hh.at[si_v.at[0]], buf, sem)

            pltpu.make_async_copy(hh.at[si_v.at[1]], buf1, sem1).wait()
            pltpu.sync_copy(buf1, acc_sh.at[di_v.at[1]], add=True)
            return carry

        lax.fori_loop(0, _CH // 2, body, 0)
        plsc.subcore_barrier()
        pltpu.sync_copy(acc_sh.at[pl.ds(r0, _RPT)],
                        out_hbm.at[c, pl.ds(r0, _RPT)])


def _dinv(deg):
    return jnp.where(deg > 0.0, lax.rsqrt(jnp.where(deg > 0.0, deg, 1.0)), 0.0)


def _mm_first_body(x_ref, w_ref, dego_ref, o_ref):
    y = jnp.dot(x_ref[...], w_ref[...], preferred_element_type=jnp.float32)
    y = y * _dinv(dego_ref[...])
    o_ref[0] = y[:, :_H]
    o_ref[1] = y[:, _H:]


def _mm_first(x, w, dego):
    return pl.pallas_call(
        _mm_first_body,
        grid=(_N // _BN,),
        in_specs=[
            pl.BlockSpec((_BN, _C), lambda i: (i, 0)),
            pl.BlockSpec((_C, _C), lambda i: (0, 0)),
            pl.BlockSpec((_BN, 1), lambda i: (i, 0)),
        ],
        out_specs=pl.BlockSpec((_NC, _BN, _H), lambda i: (0, i, 0)),
        out_shape=jax.ShapeDtypeStruct((_NC, _N, _H), jnp.float32),
    )(x, w, dego)


def _mm_mid_body(acc_ref, degi_ref, b_ref, w_ref, dego_ref, o_ref):
    x = jnp.concatenate([acc_ref[0], acc_ref[1]], axis=1)
    x = jnp.maximum(x * _dinv(degi_ref[...]) + b_ref[...], 0.0)
    y = jnp.dot(x, w_ref[...], preferred_element_type=jnp.float32)
    y = y * _dinv(dego_ref[...])
    o_ref[0] = y[:, :_H]
    o_ref[1] = y[:, _H:]


def _mm_mid(acc, degi, b, w, dego):
    return pl.pallas_call(
        _mm_mid_body,
        grid=(_N // _BN,),
        in_specs=[
            pl.BlockSpec((_NC, _BN, _H), lambda i: (0, i, 0)),
            pl.BlockSpec((_BN, 1), lambda i: (i, 0)),
            pl.BlockSpec((1, _C), lambda i: (0, 0)),
            pl.BlockSpec((_C, _C), lambda i: (0, 0)),
            pl.BlockSpec((_BN, 1), lambda i: (i, 0)),
        ],
        out_specs=pl.BlockSpec((_NC, _BN, _H), lambda i: (0, i, 0)),
        out_shape=jax.ShapeDtypeStruct((_NC, _N, _H), jnp.float32),
    )(acc, degi, b, w, dego)


def _post_body(acc_ref, deg_ref, b_ref, o_ref):
    x = jnp.concatenate([acc_ref[0], acc_ref[1]], axis=1)
    o_ref[...] = x * _dinv(deg_ref[...]) + b_ref[...]


def _post(acc, deg, b):
    return pl.pallas_call(
        _post_body,
        grid=(_N // _BN,),
        in_specs=[
            pl.BlockSpec((_NC, _BN, _H), lambda i: (0, i, 0)),
            pl.BlockSpec((_BN, 1), lambda i: (i, 0)),
            pl.BlockSpec((1, _C), lambda i: (0, 0)),
        ],
        out_specs=pl.BlockSpec((_BN, _C), lambda i: (i, 0)),
        out_shape=jax.ShapeDtypeStruct((_N, _C), jnp.float32),
    )(acc, deg, b)


def _prep_packed(ei):
    src, dst = ei[0], ei[1]
    r = jnp.arange(_EPAD - _E, dtype=jnp.int32)
    psrc = r % _N
    pdst = _N + r % (_NPAD - _N)
    packed = jnp.concatenate([
        jnp.left_shift(src, _PSH) | dst,
        jnp.left_shift(psrc, _PSH) | pdst,
    ])
    return packed.reshape(_NS, _CH, _K)


def kernel(edge_index_user_to_item, edge_index_item_rev_user,
           emb_user, emb_item,
           W1_u2i, b1_u2i, W1_i2u, b1_i2u,
           W2_u2i, b2_u2i, W2_i2u, b2_i2u):
    pk_a = _prep_packed(edge_index_user_to_item)
    pk_b = _prep_packed(edge_index_item_rev_user)
    pk_both = jnp.stack([pk_a, pk_b])

    deg_out = _deg_kernel(pk_both)
    deg_a = deg_out[0, :_N, 0:1]
    deg_b = deg_out[1, :_N, 0:1]

    b1u = b1_u2i.reshape(1, _C)
    b1i = b1_i2u.reshape(1, _C)
    b2u = b2_u2i.reshape(1, _C)
    b2i = b2_i2u.reshape(1, _C)

    h1a = _mm_first(emb_user, W1_u2i, deg_a)
    h1b = _mm_first(emb_item, W1_i2u, deg_b)
    acc1a, acc1b = _agg2_kernel(h1a, pk_a, h1b, pk_b)
    h2a = _mm_mid(acc1b, deg_b, b1i, W2_u2i, deg_a)
    h2b = _mm_mid(acc1a, deg_a, b1u, W2_i2u, deg_b)
    acc2a, acc2b = _agg2_kernel(h2a, pk_a, h2b, pk_b)

    z_item2 = _post(acc2a, deg_a, b2u)
    z_user2 = _post(acc2b, deg_b, b2i)
    return (z_user2, z_item2)


# speedup vs baseline: 1.0644x; 1.0644x over previous
import functools

import jax
import jax.numpy as jnp
from jax import lax
from jax.experimental import pallas as pl
from jax.experimental.pallas import tpu as pltpu
from jax.experimental.pallas import tpu_sc as plsc

_N = 10000
_C = 256
_E = 160000
_NC = 2
_NS = 16
_K = 128
_CH = 80
_EPAD = _NS * _CH * _K
_NPAD = 10112
_RPT = _NPAD // _NS
_H = _C // 2
_BN = 1000
_PSH = 14

_mesh = plsc.VectorSubcoreMesh(core_axis_name="c", subcore_axis_name="s")


def _fill_f32(buf, nrows, value):
    v = jnp.full((16,), value, jnp.float32)

    def body(i, carry):
        buf[i // 8, pl.ds((i % 8) * 16, 16)] = v
        return carry

    lax.fori_loop(0, nrows * 8, body, 0)


def _zero_own_rows(buf, acc_sh, r0):
    _fill_f32(buf, _K, 0.0)
    for off in range(0, _RPT - 120, _K):
        pltpu.sync_copy(buf, acc_sh.at[pl.ds(r0 + off, _K)])
    pltpu.sync_copy(buf.at[pl.ds(0, 120)],
                    acc_sh.at[pl.ds(r0 + _RPT - 120, 120)])


@functools.partial(
    pl.kernel, mesh=_mesh,
    out_type=jax.ShapeDtypeStruct((_NC, _NPAD, _H), jnp.float32),
    scratch_types=[
        pltpu.VMEM((_CH, _K), jnp.int32),
        pltpu.VMEM((1, _K), jnp.int32),
        pltpu.VMEM((_K, _H), jnp.float32),
        pltpu.VMEM_SHARED((_NPAD, _H), jnp.float32),
    ],
)
def _deg_kernel(pk_hbm, out_hbm, pk_v, di_v, ones_v, acc_sh):
    c = lax.axis_index("c")
    s = lax.axis_index("s")
    pltpu.sync_copy(pk_hbm.at[c, s], pk_v)
    r0 = s * _RPT
    _zero_own_rows(ones_v, acc_sh, r0)
    _fill_f32(ones_v, _K, 1.0)
    plsc.subcore_barrier()

    def body(j, carry):
        for u in range(8):
            sl = pl.ds(u * 16, 16)
            di_v[0, sl] = pk_v[j, sl] & ((1 << _PSH) - 1)
        pltpu.sync_copy(ones_v, acc_sh.at[di_v.at[0]], add=True)
        return carry

    lax.fori_loop(0, _CH, body, 0)
    plsc.subcore_barrier()
    pltpu.sync_copy(acc_sh.at[pl.ds(r0, _RPT)], out_hbm.at[c, pl.ds(r0, _RPT)])


@functools.partial(
    pl.kernel, mesh=_mesh,
    out_type=jax.ShapeDtypeStruct((_NC, _NPAD, _H), jnp.float32),
    scratch_types=[
        pltpu.VMEM((_CH, _K), jnp.int32),
        pltpu.VMEM((2, _K), jnp.int32),
        pltpu.VMEM((2, _K), jnp.int32),
        pltpu.VMEM((_K, _H), jnp.float32),
        pltpu.VMEM((_K, _H), jnp.float32),
        pltpu.VMEM_SHARED((_NPAD, _H), jnp.float32),
        pltpu.SemaphoreType.DMA,
        pltpu.SemaphoreType.DMA,
    ],
)
def _agg_kernel(h_hbm, pk_hbm, out_hbm,
                pk_v, si_v, di_v, buf, buf1, acc_sh, sem, sem1):
    c = lax.axis_index("c")
    s = lax.axis_index("s")
    pltpu.sync_copy(pk_hbm.at[s], pk_v)
    r0 = s * _RPT
    hh = h_hbm.at[c]
    _zero_own_rows(buf, acc_sh, r0)
    plsc.subcore_barrier()

    def unpack(j, slot):
        for u in range(8):
            sl = pl.ds(u * 16, 16)
            pk = pk_v[j, sl]
            si_v[slot, sl] = lax.shift_right_logical(pk, _PSH)
            di_v[slot, sl] = pk & ((1 << _PSH) - 1)

    unpack(0, 0)
    pltpu.async_copy(hh.at[si_v.at[0]], buf, sem)

    def body(t, carry):
        g = 2 * t
        unpack(g + 1, 1)
        pltpu.async_copy(hh.at[si_v.at[1]], buf1, sem1)
        pltpu.make_async_copy(hh.at[si_v.at[0]], buf, sem).wait()
        pltpu.sync_copy(buf, acc_sh.at[di_v.at[0]], add=True)

        @pl.when(g + 2 < _CH)
        def _():
            unpack(g + 2, 0)
            pltpu.async_copy(hh.at[si_v.at[0]], buf, sem)

        pltpu.make_async_copy(hh.at[si_v.at[1]], buf1, sem1).wait()
        pltpu.sync_copy(buf1, acc_sh.at[di_v.at[1]], add=True)
        return carry

    lax.fori_loop(0, _CH // 2, body, 0)
    plsc.subcore_barrier()
    pltpu.sync_copy(acc_sh.at[pl.ds(r0, _RPT)],
                    out_hbm.at[c, pl.ds(r0, _RPT)])


def _dinv(deg):
    return jnp.where(deg > 0.0, lax.rsqrt(jnp.where(deg > 0.0, deg, 1.0)), 0.0)


def _mm_first_body(x_ref, w_ref, dego_ref, o_ref):
    y = jnp.dot(x_ref[...], w_ref[...], preferred_element_type=jnp.float32)
    y = y * _dinv(dego_ref[...])
    o_ref[0] = y[:, :_H]
    o_ref[1] = y[:, _H:]


def _mm_first(x, w, dego):
    return pl.pallas_call(
        _mm_first_body,
        grid=(_N // _BN,),
        in_specs=[
            pl.BlockSpec((_BN, _C), lambda i: (i, 0)),
            pl.BlockSpec((_C, _C), lambda i: (0, 0)),
            pl.BlockSpec((_BN, 1), lambda i: (i, 0)),
        ],
        out_specs=pl.BlockSpec((_NC, _BN, _H), lambda i: (0, i, 0)),
        out_shape=jax.ShapeDtypeStruct((_NC, _N, _H), jnp.float32),
    )(x, w, dego)


def _mm_mid_body(acc_ref, degi_ref, b_ref, w_ref, dego_ref, o_ref):
    x = jnp.concatenate([acc_ref[0], acc_ref[1]], axis=1)
    x = jnp.maximum(x * _dinv(degi_ref[...]) + b_ref[...], 0.0)
    y = jnp.dot(x, w_ref[...], preferred_element_type=jnp.float32)
    y = y * _dinv(dego_ref[...])
    o_ref[0] = y[:, :_H]
    o_ref[1] = y[:, _H:]


def _mm_mid(acc, degi, b, w, dego):
    return pl.pallas_call(
        _mm_mid_body,
        grid=(_N // _BN,),
        in_specs=[
            pl.BlockSpec((_NC, _BN, _H), lambda i: (0, i, 0)),
            pl.BlockSpec((_BN, 1), lambda i: (i, 0)),
            pl.BlockSpec((1, _C), lambda i: (0, 0)),
            pl.BlockSpec((_C, _C), lambda i: (0, 0)),
            pl.BlockSpec((_BN, 1), lambda i: (i, 0)),
        ],
        out_specs=pl.BlockSpec((_NC, _BN, _H), lambda i: (0, i, 0)),
        out_shape=jax.ShapeDtypeStruct((_NC, _N, _H), jnp.float32),
    )(acc, degi, b, w, dego)


def _post_body(acc_ref, deg_ref, b_ref, o_ref):
    x = jnp.concatenate([acc_ref[0], acc_ref[1]], axis=1)
    o_ref[...] = x * _dinv(deg_ref[...]) + b_ref[...]


def _post(acc, deg, b):
    return pl.pallas_call(
        _post_body,
        grid=(_N // _BN,),
        in_specs=[
            pl.BlockSpec((_NC, _BN, _H), lambda i: (0, i, 0)),
            pl.BlockSpec((_BN, 1), lambda i: (i, 0)),
            pl.BlockSpec((1, _C), lambda i: (0, 0)),
        ],
        out_specs=pl.BlockSpec((_BN, _C), lambda i: (i, 0)),
        out_shape=jax.ShapeDtypeStruct((_N, _C), jnp.float32),
    )(acc, deg, b)


def _prep_packed(ei):
    src, dst = ei[0], ei[1]
    r = jnp.arange(_EPAD - _E, dtype=jnp.int32)
    psrc = r % _N
    pdst = _N + r % (_NPAD - _N)
    packed = jnp.concatenate([
        jnp.left_shift(src, _PSH) | dst,
        jnp.left_shift(psrc, _PSH) | pdst,
    ])
    return packed.reshape(_NS, _CH, _K)


def kernel(edge_index_user_to_item, edge_index_item_rev_user,
           emb_user, emb_item,
           W1_u2i, b1_u2i, W1_i2u, b1_i2u,
           W2_u2i, b2_u2i, W2_i2u, b2_i2u):
    pk_a = _prep_packed(edge_index_user_to_item)
    pk_b = _prep_packed(edge_index_item_rev_user)
    pk_both = jnp.stack([pk_a, pk_b])

    deg_out = _deg_kernel(pk_both)
    deg_a = deg_out[0, :_N, 0:1]
    deg_b = deg_out[1, :_N, 0:1]

    b1u = b1_u2i.reshape(1, _C)
    b1i = b1_i2u.reshape(1, _C)
    b2u = b2_u2i.reshape(1, _C)
    b2i = b2_i2u.reshape(1, _C)

    h1a = _mm_first(emb_user, W1_u2i, deg_a)
    acc1a = _agg_kernel(h1a, pk_a)
    h1b = _mm_first(emb_item, W1_i2u, deg_b)
    acc1b = _agg_kernel(h1b, pk_b)
    h2a = _mm_mid(acc1b, deg_b, b1i, W2_u2i, deg_a)
    acc2a = _agg_kernel(h2a, pk_a)
    h2b = _mm_mid(acc1a, deg_a, b1u, W2_i2u, deg_b)
    acc2b = _agg_kernel(h2b, pk_b)

    z_item2 = _post(acc2a, deg_a, b2u)
    z_user2 = _post(acc2b, deg_b, b2i)
    return (z_user2, z_item2)
